# initial kernel scaffold (unmeasured)
import jax
import jax.numpy as jnp
from jax import lax
from jax.experimental import pallas as pl
from jax.experimental.pallas import tpu as pltpu

N_DEV = 32


def kernel(x, w_mat, scale_x, scale_w):
    m_global, k_sh = x.shape
    _, n = w_mat.shape
    m_blk = m_global // N_DEV
    s2 = (scale_x * scale_w).reshape(1, 1)

    def body(x_ref, w_ref, s_ref, out_ref, send_ref, recv_ref, send_sems, recv_sems):
        my = lax.axis_index("i")
        left = lax.rem(my + N_DEV - 1, N_DEV)
        right = lax.rem(my + 1, N_DEV)

        barrier_sem = pltpu.get_barrier_semaphore()
        for nbr in (left, right):
            pl.semaphore_signal(
                barrier_sem, inc=1,
                device_id=(nbr,), device_id_type=pl.DeviceIdType.MESH,
            )
        pl.semaphore_wait(barrier_sem, 2)

        w = w_ref[:, :]

        def partial(c):
            xb = x_ref[pl.ds(c * m_blk, m_blk), :]
            return lax.dot_general(
                xb, w, (((1,), (0,)), ((), ())),
                preferred_element_type=jnp.int32,
            )

        for h in range(N_DEV - 1):
            c = lax.rem(my + N_DEV - h - 1, N_DEV)
            p = partial(c)
            if h == 0:
                send_ref[h % 2] = p
            else:
                send_ref[h % 2] = recv_ref[h - 1] + p
            rdma = pltpu.make_async_remote_copy(
                src_ref=send_ref.at[h % 2],
                dst_ref=recv_ref.at[h],
                send_sem=send_sems.at[h % 2],
                recv_sem=recv_sems.at[h],
                device_id=(right,),
                device_id_type=pl.DeviceIdType.MESH,
            )
            rdma.start()
            rdma.wait()

        acc = recv_ref[N_DEV - 2] + partial(my)
        out_ref[:, :] = jnp.maximum(acc.astype(jnp.float32) * s_ref[0, 0], 0.0)

    return pl.pallas_call(
        body,
        out_shape=jax.ShapeDtypeStruct((m_blk, n), jnp.float32),
        in_specs=[
            pl.BlockSpec(memory_space=pltpu.VMEM),
            pl.BlockSpec(memory_space=pltpu.VMEM),
            pl.BlockSpec(memory_space=pltpu.SMEM),
        ],
        out_specs=pl.BlockSpec(memory_space=pltpu.VMEM),
        scratch_shapes=[
            pltpu.VMEM((2, m_blk, n), jnp.int32),
            pltpu.VMEM((N_DEV - 1, m_blk, n), jnp.int32),
            pltpu.SemaphoreType.DMA((2,)),
            pltpu.SemaphoreType.DMA((N_DEV - 1,)),
        ],
        compiler_params=pltpu.CompilerParams(collective_id=0),
    )(x, w_mat, s2)


# baseline (device time: 424314 ns/iter reference)
import jax
import jax.numpy as jnp
from jax import lax
from jax.experimental import pallas as pl
from jax.experimental.pallas import tpu as pltpu

N_DEV = 32


def kernel(x, w_mat, scale_x, scale_w):
    m_global, k_sh = x.shape
    _, n = w_mat.shape
    m_blk = m_global // N_DEV
    s2 = (scale_x * scale_w).reshape(1, 1)

    def body(x_ref, w_ref, s_ref, out_ref, send_ref, recv_ref, send_sems, recv_sems):
        my = lax.axis_index("i")
        left = lax.rem(my + N_DEV - 1, N_DEV)
        right = lax.rem(my + 1, N_DEV)

        barrier_sem = pltpu.get_barrier_semaphore()
        for nbr in (left, right):
            pl.semaphore_signal(
                barrier_sem, inc=1,
                device_id=(nbr,), device_id_type=pl.DeviceIdType.MESH,
            )
        pl.semaphore_wait(barrier_sem, 2)

        w = w_ref[:, :]

        def partial(c):
            xb = x_ref[pl.ds(c * m_blk, m_blk), :]
            return lax.dot_general(
                xb, w, (((1,), (0,)), ((), ())),
                preferred_element_type=jnp.int32,
            )

        for h in range(N_DEV - 1):
            c = lax.rem(my + N_DEV - h - 1, N_DEV)
            p = partial(c)
            if h == 0:
                send_ref[h % 2] = p
            else:
                send_ref[h % 2] = recv_ref[h - 1] + p
            rdma = pltpu.make_async_remote_copy(
                src_ref=send_ref.at[h % 2],
                dst_ref=recv_ref.at[h],
                send_sem=send_sems.at[h % 2],
                recv_sem=recv_sems.at[h],
                device_id=(right,),
                device_id_type=pl.DeviceIdType.MESH,
            )
            rdma.start()
            rdma.wait()

        acc = recv_ref[N_DEV - 2] + partial(my)
        out_ref[:, :] = jnp.maximum(acc.astype(jnp.float32) * s_ref[0, 0], 0.0)

    return pl.pallas_call(
        body,
        out_shape=jax.ShapeDtypeStruct((m_blk, n), jnp.float32),
        in_specs=[
            pl.BlockSpec(memory_space=pltpu.VMEM),
            pl.BlockSpec(memory_space=pltpu.VMEM),
            pl.BlockSpec(memory_space=pltpu.SMEM),
        ],
        out_specs=pl.BlockSpec(memory_space=pltpu.VMEM),
        scratch_shapes=[
            pltpu.VMEM((2, m_blk, n), jnp.int32),
            pltpu.VMEM((N_DEV - 1, m_blk, n), jnp.int32),
            pltpu.SemaphoreType.DMA((2,)),
            pltpu.SemaphoreType.DMA((N_DEV - 1,)),
        ],
        compiler_params=pltpu.CompilerParams(
            collective_id=0, vmem_limit_bytes=64 * 1024 * 1024
        ),
    )(x, w_mat, s2)


# device time: 186019 ns/iter; 2.2810x vs baseline; 2.2810x over previous
import jax
import jax.numpy as jnp
import numpy as np
from jax import lax
from jax.experimental import pallas as pl
from jax.experimental.pallas import tpu as pltpu

N_DEV = 32
N_STREAMS = 4


def _ring_tables():
    yz = [(0, 0), (0, 1), (0, 2), (0, 3), (1, 3), (1, 2), (1, 1), (2, 1),
          (2, 2), (2, 3), (3, 3), (3, 2), (3, 1), (3, 0), (2, 0), (1, 0)]
    plane = {(0, 0): 0, (1, 0): 1, (1, 1): 2, (0, 1): 3,
             (0, 2): 4, (1, 2): 5, (1, 3): 6, (0, 3): 7}
    coords = []
    for i, (y, z) in enumerate(yz):
        for x in ((0, 1) if i % 2 == 0 else (1, 0)):
            coords.append((x, y, z))
    assert all(
        sum(abs(u - v) for u, v in zip(coords[r], coords[(r + 1) % 32])) == 1
        for r in range(32)
    )
    perm = [z * 8 + plane[(x, y)] for (x, y, z) in coords]
    inv = [0] * N_DEV
    for r, m in enumerate(perm):
        inv[m] = r
    return np.array(perm, np.int32), np.array(inv, np.int32)


_PERM, _INV = _ring_tables()


def kernel(x, w_mat, scale_x, scale_w):
    m_global, k_sh = x.shape
    _, n = w_mat.shape
    m_blk = m_global // N_DEV
    half = n // 2
    sub = n // N_STREAMS
    s2 = (scale_x * scale_w).reshape(1, 1)

    def body(x_ref, w_ref, s_ref, perm_ref, inv_ref, out_ref,
             send_ref, recv_ref, send_sems, recv_sems):
        my = lax.axis_index("i")
        rp = inv_ref[my]
        right = perm_ref[lax.rem(rp + 1, N_DEV)]
        left = perm_ref[lax.rem(rp + N_DEV - 1, N_DEV)]

        barrier_sem = pltpu.get_barrier_semaphore()
        for nbr in (left, right):
            pl.semaphore_signal(
                barrier_sem, inc=1,
                device_id=(nbr,), device_id_type=pl.DeviceIdType.MESH,
            )
        pl.semaphore_wait(barrier_sem, 2)

        def rdma(s, h, tgt):
            return pltpu.make_async_remote_copy(
                src_ref=send_ref.at[s, h % 2],
                dst_ref=recv_ref.at[s, h],
                send_sem=send_sems.at[s, h % 2],
                recv_sem=recv_sems.at[s, h],
                device_id=(tgt,),
                device_id_type=pl.DeviceIdType.MESH,
            )

        def partial(c, col0, ncol):
            xb = x_ref[pl.ds(c * m_blk, m_blk), :]
            return lax.dot_general(
                xb, w_ref[:, col0:col0 + ncol],
                (((1,), (0,)), ((), ())),
                preferred_element_type=jnp.int32,
            )

        for h in range(N_DEV - 1):
            c_cw = perm_ref[lax.rem(rp + N_DEV - h - 1, N_DEV)]
            c_ccw = perm_ref[lax.rem(rp + h + 1, N_DEV)]
            p_cw = partial(c_cw, 0, half)
            p_ccw = partial(c_ccw, half, half)
            for s, tgt, p, c0 in (
                (0, right, p_cw, 0),
                (1, right, p_cw, sub),
                (2, left, p_ccw, 0),
                (3, left, p_ccw, sub),
            ):
                if h >= 2:
                    rdma(s, h - 2, tgt).wait_send()
                if h == 0:
                    val = p[:, c0:c0 + sub]
                else:
                    rdma(s, h - 1, tgt).wait_recv()
                    val = recv_ref[s, h - 1] + p[:, c0:c0 + sub]
                send_ref[s, h % 2] = val
                rdma(s, h, tgt).start()

        pm = partial(my, 0, n)
        scale = s_ref[0, 0]
        for s in range(N_STREAMS):
            tgt = right if s < 2 else left
            rdma(s, N_DEV - 2, tgt).wait_recv()
            acc = recv_ref[s, N_DEV - 2] + pm[:, s * sub:(s + 1) * sub]
            out_ref[:, s * sub:(s + 1) * sub] = jnp.maximum(
                acc.astype(jnp.float32) * scale, 0.0
            )
        for s in range(N_STREAMS):
            tgt = right if s < 2 else left
            rdma(s, N_DEV - 3, tgt).wait_send()
            rdma(s, N_DEV - 2, tgt).wait_send()

    return pl.pallas_call(
        body,
        out_shape=jax.ShapeDtypeStruct((m_blk, n), jnp.float32),
        in_specs=[
            pl.BlockSpec(memory_space=pltpu.VMEM),
            pl.BlockSpec(memory_space=pltpu.VMEM),
            pl.BlockSpec(memory_space=pltpu.SMEM),
            pl.BlockSpec(memory_space=pltpu.SMEM),
            pl.BlockSpec(memory_space=pltpu.SMEM),
        ],
        out_specs=pl.BlockSpec(memory_space=pltpu.VMEM),
        scratch_shapes=[
            pltpu.VMEM((N_STREAMS, 2, m_blk, sub), jnp.int32),
            pltpu.VMEM((N_STREAMS, N_DEV - 1, m_blk, sub), jnp.int32),
            pltpu.SemaphoreType.DMA((N_STREAMS, 2)),
            pltpu.SemaphoreType.DMA((N_STREAMS, N_DEV - 1)),
        ],
        compiler_params=pltpu.CompilerParams(
            collective_id=0, vmem_limit_bytes=64 * 1024 * 1024
        ),
    )(x, w_mat, s2, jnp.asarray(_PERM), jnp.asarray(_INV))
